# Initial kernel scaffold; baseline (speedup 1.0000x reference)
import jax
import jax.numpy as jnp
from jax.experimental import pallas as pl


def kernel(x, edge_index, W1, b1, g1, be1, W2, b2, g2, be2):
    raise NotImplementedError("write your pallas kernel here")


# speedup vs baseline: 1.1792x; 1.1792x over previous
import functools

import jax
import jax.numpy as jnp
from jax import lax
from jax.experimental import pallas as pl
from jax.experimental.pallas import tpu as pltpu
from jax.experimental.pallas import tpu_sc as plsc

N = 10000
E = 320000
D = 128
EPS = 1e-5

NC = 2
NS = 16
NW = NC * NS

CHUNK = 640
CROWS = CHUNK // 128
NEG = -3.0e38

_f32 = jnp.float32
_i32 = jnp.int32

_MESH = plsc.VectorSubcoreMesh(core_axis_name="c", subcore_axis_name="s")


def _mm_body(x_ref, wa_ref, wb_ref, xa_ref, xb_ref):
    xv = x_ref[...]
    xa_ref[...] = jnp.dot(xv, wa_ref[...], preferred_element_type=_f32)
    xb_ref[...] = jnp.dot(xv, wb_ref[...], preferred_element_type=_f32)


def _precompute(x, W):
    wa = W[:D, :] - W[D:, :]
    wb = W[D:, :]
    return pl.pallas_call(
        _mm_body,
        out_shape=(
            jax.ShapeDtypeStruct((N, D), _f32),
            jax.ShapeDtypeStruct((N, D), _f32),
        ),
    )(x, wa, wb)


_EPW_B = E // 4
_NCH_B = _EPW_B // CHUNK


@functools.partial(
    pl.kernel,
    out_type=jax.ShapeDtypeStruct((4, 8, 2, 16), _f32),
    mesh=_MESH,
    compiler_params=pltpu.CompilerParams(use_tc_tiling_on_sc=False, needs_layout_passes=False),
    scratch_types=[
        pltpu.VMEM((CHUNK,), _i32),
        pltpu.VMEM((CHUNK,), _i32),
        pltpu.VMEM((CROWS, 128), _i32),
        pltpu.VMEM((CROWS, 128), _i32),
        pltpu.VMEM((CHUNK, 16), _f32),
        pltpu.VMEM((CHUNK, 16), _f32),
        pltpu.VMEM((2, 16), _f32),
        pltpu.SemaphoreType.DMA,
        pltpu.SemaphoreType.DMA,
    ],
)
def _stats_kernel(xa8, xb8, srcs, dsts, out, s_idx, d_idx, gia, gib, ra, rb,
                  ost, sema, semb):
    wid = lax.axis_index("s") * NC + lax.axis_index("c")
    sh_id = wid // 8
    g_id = wid % 8
    base = sh_id * _EPW_B

    def chunk_body(ci, carry):
        sh, sh2 = carry
        off = base + ci * CHUNK
        pltpu.sync_copy(srcs.at[pl.ds(off, CHUNK)], s_idx)
        pltpu.sync_copy(dsts.at[pl.ds(off, CHUNK)], d_idx)

        def idx_body(j, _):
            r = j // 8
            c = (j % 8) * 16
            dv = d_idx[pl.ds(j * 16, 16)]
            gia[r, pl.ds(c, 16)] = dv * 8 + g_id
            sv = s_idx[pl.ds(j * 16, 16)]
            gib[r, pl.ds(c, 16)] = sv * 8 + g_id
            return 0

        lax.fori_loop(0, CROWS * 8, idx_body, 0, unroll=True)

        cps = []
        for r in range(CROWS):
            cps.append(pltpu.async_copy(
                xa8.at[gia.at[r]], ra.at[pl.ds(r * 128, 128), :], sema))
            cps.append(pltpu.async_copy(
                xb8.at[gib.at[r]], rb.at[pl.ds(r * 128, 128), :], semb))
        for cp in cps:
            cp.wait()

        def acc_body(i, c):
            csh, csh2 = c
            t = ra[i, :] + rb[i, :]
            return (csh + t, csh2 + t * t)

        z = jnp.zeros((16,), _f32)
        csh, csh2 = lax.fori_loop(0, CHUNK, acc_body, (z, z))
        return (sh + csh, sh2 + csh2)

    z = jnp.zeros((16,), _f32)
    sh, sh2 = lax.fori_loop(0, _NCH_B, chunk_body, (z, z))
    ost[0, :] = sh
    ost[1, :] = sh2
    pltpu.sync_copy(ost, out.at[sh_id, g_id])


def _scale_body2(ss_ref, b_ref, g_ref, be_ref, xb_ref, xa_ref,
                 xbs_ref, xas_ref, cv_ref):
    ss = ss_ref[...]
    sh = jnp.sum(ss[:, 0, :], axis=0, keepdims=True)
    sh2 = jnp.sum(ss[:, 1, :], axis=0, keepdims=True)
    b = b_ref[...]
    g = g_ref[...]
    be = be_ref[...]
    mean = (sh + E * b) / E
    msq = (sh2 + 2.0 * b * sh + E * b * b) / E
    var = msq - mean * mean
    s = g / jnp.sqrt(var + EPS)
    cv_ref[...] = s * (b - mean) + be
    xbs_ref[...] = xb_ref[...] * s
    xas_ref[...] = xa_ref[...] * s


def _scale2(ss, b, g, be, xb, xa):
    return pl.pallas_call(
        _scale_body2,
        out_shape=(
            jax.ShapeDtypeStruct((N, D), _f32),
            jax.ShapeDtypeStruct((N, D), _f32),
            jax.ShapeDtypeStruct((1, D), _f32),
        ),
    )(ss, b.reshape(1, D), g.reshape(1, D), be.reshape(1, D), xb, xa)


_EPW_D = E // 2
_NCH_D = _EPW_D // CHUNK
_NSTEP = CHUNK // 2


@functools.partial(
    pl.kernel,
    out_type=jax.ShapeDtypeStruct((2, 16, N * 8), _f32),
    mesh=_MESH,
    compiler_params=pltpu.CompilerParams(use_tc_tiling_on_sc=False, needs_layout_passes=False),
    scratch_types=[
        pltpu.VMEM((CHUNK,), _i32),
        pltpu.VMEM((CHUNK,), _i32),
        pltpu.VMEM((CROWS, 128), _i32),
        pltpu.VMEM((CHUNK, 8), _f32),
        pltpu.VMEM((N * 8,), _f32),
        pltpu.SemaphoreType.DMA,
    ],
)
def _segmax_kernel(xbs16, srcs, dsts, out, s_idx, d_idx, gi, rows, acc, sem):
    wid = lax.axis_index("s") * NC + lax.axis_index("c")
    sh_id = wid // 16
    g_id = wid % 16
    base = sh_id * _EPW_D

    io = lax.iota(_i32, 16)
    k7 = io & 7
    khigh = io >> 3
    klow = 1 - khigh

    neg = jnp.full((16,), NEG, _f32)

    def init_body(i, _):
        acc[pl.ds(i * 16, 16)] = neg
        return 0

    lax.fori_loop(0, N * 8 // 16, init_body, 0)

    def chunk_body(ci, _):
        off = base + ci * CHUNK
        pltpu.sync_copy(srcs.at[pl.ds(off, CHUNK)], s_idx)
        pltpu.sync_copy(dsts.at[pl.ds(off, CHUNK)], d_idx)

        def idx_body(j, _):
            r = j // 8
            c = (j % 8) * 16
            sv = s_idx[pl.ds(j * 16, 16)]
            gi[r, pl.ds(c, 16)] = sv * 16 + g_id
            return 0

        lax.fori_loop(0, CROWS * 8, idx_body, 0, unroll=True)

        cps = []
        for r in range(CROWS):
            cps.append(pltpu.async_copy(
                xbs16.at[gi.at[r]], rows.at[pl.ds(r * 128, 128), :], sem))
        for cp in cps:
            cp.wait()

        def step_body(i, _):
            sel = 2 * i + khigh
            selp = 2 * i + klow
            dstv = plsc.load_gather(d_idx, [sel])
            dstvp = plsc.load_gather(d_idx, [selp])
            collide = dstv == dstvp
            v = plsc.load_gather(rows, [sel, k7])
            vp = plsc.load_gather(rows, [selp, k7])
            v2 = jnp.where(collide, jnp.maximum(v, vp), v)
            aidx = dstv * 8 + k7
            cur = plsc.load_gather(acc, [aidx])
            plsc.store_scatter(acc, [aidx], jnp.maximum(cur, v2))
            return 0

        lax.fori_loop(0, _NSTEP, step_body, 0)
        return 0

    lax.fori_loop(0, _NCH_D, chunk_body, 0)
    pltpu.sync_copy(acc, out.at[sh_id, g_id])


def _assemble_body(x_ref, xas_ref, cv_ref, m0_ref, m1_ref, out_ref):
    m = jnp.maximum(m0_ref[...], m1_ref[...])
    h = xas_ref[...] + cv_ref[...] + m
    out_ref[...] = jnp.maximum(h, 0.0) + x_ref[...]


def _assemble(x, xas, cv, m0, m1):
    return pl.pallas_call(
        _assemble_body,
        out_shape=jax.ShapeDtypeStruct((N, D), _f32),
    )(x, xas, cv, m0, m1)


def _block(x, srcs, dsts, W, b, g, be):
    xa, xb = _precompute(x, W)
    xa8 = xa.reshape(N * 8, 16)
    xb8 = xb.reshape(N * 8, 16)
    ss = _stats_kernel(xa8, xb8, srcs, dsts)
    ss = ss.reshape(4, 8, 2, 16).transpose(0, 2, 1, 3).reshape(4, 2, D)
    xbs, xas, cv = _scale2(ss, b, g, be, xb, xa)
    xbs16 = xbs.reshape(N * 16, 8)
    mm = _segmax_kernel(xbs16, srcs, dsts)
    mm = mm.reshape(2, 16, N, 8).transpose(0, 2, 1, 3).reshape(2, N, D)
    return _assemble(x, xas, cv, mm[0], mm[1])


def kernel(x, edge_index, W1, b1, g1, be1, W2, b2, g2, be2):
    srcs = edge_index[0]
    dsts = edge_index[1]
    x1 = _block(x, srcs, dsts, W1, b1, g1, be1)
    x2 = _block(x1, srcs, dsts, W2, b2, g2, be2)
    return x2


# speedup vs baseline: 1.6435x; 1.3938x over previous
import functools

import jax
import jax.numpy as jnp
from jax import lax
from jax.experimental import pallas as pl
from jax.experimental.pallas import tpu as pltpu
from jax.experimental.pallas import tpu_sc as plsc

N = 10000
E = 320000
D = 128
EPS = 1e-5

NC = 2
NS = 16

CHUNK = 640
CROWS = CHUNK // 128
NEG = -3.0e38

_f32 = jnp.float32
_i32 = jnp.int32

_MESH = plsc.VectorSubcoreMesh(core_axis_name="c", subcore_axis_name="s")
_SC_PARAMS = pltpu.CompilerParams(use_tc_tiling_on_sc=False,
                                  needs_layout_passes=False)


def _mm_body(x_ref, wa_ref, wb_ref, xa_ref, xb_ref):
    xv = x_ref[...]
    xa_ref[...] = jnp.dot(xv, wa_ref[...], preferred_element_type=_f32)
    xb_ref[...] = jnp.dot(xv, wb_ref[...], preferred_element_type=_f32)


def _precompute(x, W):
    wa = W[:D, :] - W[D:, :]
    wb = W[D:, :]
    return pl.pallas_call(
        _mm_body,
        out_shape=(
            jax.ShapeDtypeStruct((N, D), _f32),
            jax.ShapeDtypeStruct((N, D), _f32),
        ),
    )(x, wa, wb)


_EPW_B = E // 4
_NCH_B = _EPW_B // CHUNK


@functools.partial(
    pl.kernel,
    out_type=jax.ShapeDtypeStruct((4, 8, 2, 16), _f32),
    mesh=_MESH,
    compiler_params=_SC_PARAMS,
    scratch_types=[
        pltpu.VMEM((2, CHUNK), _i32),
        pltpu.VMEM((2, CHUNK), _i32),
        pltpu.VMEM((2, CROWS, 128), _i32),
        pltpu.VMEM((2, CROWS, 128), _i32),
        pltpu.VMEM((2, CHUNK, 16), _f32),
        pltpu.VMEM((2, CHUNK, 16), _f32),
        pltpu.VMEM((2, 16), _f32),
        pltpu.SemaphoreType.DMA((2,)),
        pltpu.SemaphoreType.DMA((2,)),
    ],
)
def _stats_kernel(xa8, xb8, srcs, dsts, out, s_idx, d_idx, gia, gib, ra, rb,
                  accs, isem, gsem):
    wid = lax.axis_index("s") * NC + lax.axis_index("c")
    sh_id = wid // 8
    g_id = wid % 8
    base = sh_id * _EPW_B

    accs[0, :] = jnp.zeros((16,), _f32)
    accs[1, :] = jnp.zeros((16,), _f32)

    def issue(ci, P):
        off = base + ci * CHUNK
        pltpu.async_copy(srcs.at[pl.ds(off, CHUNK)], s_idx.at[P], isem.at[P])
        pltpu.async_copy(dsts.at[pl.ds(off, CHUNK)], d_idx.at[P], isem.at[P])

    def launch(P):
        pltpu.make_async_copy(srcs.at[pl.ds(0, CHUNK)], s_idx.at[P],
                              isem.at[P]).wait()
        pltpu.make_async_copy(dsts.at[pl.ds(0, CHUNK)], d_idx.at[P],
                              isem.at[P]).wait()

        def idx_body(j, _):
            r = j // 8
            c = (j % 8) * 16
            dv = d_idx[P, pl.ds(j * 16, 16)]
            gia[P, r, pl.ds(c, 16)] = dv * 8 + g_id
            sv = s_idx[P, pl.ds(j * 16, 16)]
            gib[P, r, pl.ds(c, 16)] = sv * 8 + g_id
            return 0

        lax.fori_loop(0, CROWS * 8, idx_body, 0, unroll=True)
        for r in range(CROWS):
            pltpu.async_copy(xa8.at[gia.at[P, r]],
                             ra.at[P, pl.ds(r * 128, 128), :], gsem.at[P])
            pltpu.async_copy(xb8.at[gib.at[P, r]],
                             rb.at[P, pl.ds(r * 128, 128), :], gsem.at[P])

    def compute(P):
        for r in range(CROWS):
            pltpu.make_async_copy(xa8.at[gia.at[P, r]],
                                  ra.at[P, pl.ds(r * 128, 128), :],
                                  gsem.at[P]).wait()
            pltpu.make_async_copy(xb8.at[gib.at[P, r]],
                                  rb.at[P, pl.ds(r * 128, 128), :],
                                  gsem.at[P]).wait()

        def acc_body(i, c):
            csh, csh2 = c
            t = ra[P, i, :] + rb[P, i, :]
            return (csh + t, csh2 + t * t)

        z = jnp.zeros((16,), _f32)
        csh, csh2 = lax.fori_loop(0, CHUNK, acc_body, (z, z))
        accs[0, :] = accs[0, :] + csh
        accs[1, :] = accs[1, :] + csh2

    issue(0, 0)
    launch(0)
    issue(1, 1)

    def pair_body(k, _):
        c1 = 2 * k + 1
        c2 = 2 * k + 2
        c3 = 2 * k + 3

        @pl.when(c1 < _NCH_B)
        def _():
            launch(1)

        compute(0)

        @pl.when(c2 < _NCH_B)
        def _():
            issue(c2, 0)
            launch(0)

        @pl.when(c1 < _NCH_B)
        def _():
            compute(1)

        @pl.when(c3 < _NCH_B)
        def _():
            issue(c3, 1)

        return 0

    lax.fori_loop(0, (_NCH_B + 1) // 2, pair_body, 0)
    pltpu.sync_copy(accs, out.at[sh_id, g_id])


def _scale_body2(ss_ref, b_ref, g_ref, be_ref, xb_ref, xa_ref,
                 xbs_ref, xas_ref, cv_ref):
    ss = ss_ref[...]
    sh = jnp.sum(ss[:, 0, :], axis=0, keepdims=True)
    sh2 = jnp.sum(ss[:, 1, :], axis=0, keepdims=True)
    b = b_ref[...]
    g = g_ref[...]
    be = be_ref[...]
    mean = (sh + E * b) / E
    msq = (sh2 + 2.0 * b * sh + E * b * b) / E
    var = msq - mean * mean
    s = g / jnp.sqrt(var + EPS)
    cv_ref[...] = s * (b - mean) + be
    xbs_ref[...] = xb_ref[...] * s
    xas_ref[...] = xa_ref[...] * s


def _scale2(ss, b, g, be, xb, xa):
    return pl.pallas_call(
        _scale_body2,
        out_shape=(
            jax.ShapeDtypeStruct((N, D), _f32),
            jax.ShapeDtypeStruct((N, D), _f32),
            jax.ShapeDtypeStruct((1, D), _f32),
        ),
    )(ss, b.reshape(1, D), g.reshape(1, D), be.reshape(1, D), xb, xa)


_EPW_D = E // 2
_NCH_D = _EPW_D // CHUNK
_NSTEP = CHUNK // 2


@functools.partial(
    pl.kernel,
    out_type=jax.ShapeDtypeStruct((2, 16, N * 8), _f32),
    mesh=_MESH,
    compiler_params=_SC_PARAMS,
    scratch_types=[
        pltpu.VMEM((2, CHUNK), _i32),
        pltpu.VMEM((2, CHUNK), _i32),
        pltpu.VMEM((2, CROWS, 128), _i32),
        pltpu.VMEM((2, CHUNK, 8), _f32),
        pltpu.VMEM((N * 8,), _f32),
        pltpu.SemaphoreType.DMA((2,)),
        pltpu.SemaphoreType.DMA((2,)),
    ],
)
def _segmax_kernel(xbs16, srcs, dsts, out, s_idx, d_idx, gi, rows, acc,
                   isem, gsem):
    wid = lax.axis_index("s") * NC + lax.axis_index("c")
    sh_id = wid // 16
    g_id = wid % 16
    base = sh_id * _EPW_D

    io = lax.iota(_i32, 16)
    k7 = io & 7
    khigh = io >> 3
    klow = 1 - khigh

    neg = jnp.full((16,), NEG, _f32)

    def init_body(i, _):
        acc[pl.ds(i * 16, 16)] = neg
        return 0

    lax.fori_loop(0, N * 8 // 16, init_body, 0)

    def issue(ci, P):
        off = base + ci * CHUNK
        pltpu.async_copy(srcs.at[pl.ds(off, CHUNK)], s_idx.at[P], isem.at[P])
        pltpu.async_copy(dsts.at[pl.ds(off, CHUNK)], d_idx.at[P], isem.at[P])

    def launch(P):
        pltpu.make_async_copy(srcs.at[pl.ds(0, CHUNK)], s_idx.at[P],
                              isem.at[P]).wait()
        pltpu.make_async_copy(dsts.at[pl.ds(0, CHUNK)], d_idx.at[P],
                              isem.at[P]).wait()

        def idx_body(j, _):
            r = j // 8
            c = (j % 8) * 16
            sv = s_idx[P, pl.ds(j * 16, 16)]
            gi[P, r, pl.ds(c, 16)] = sv * 16 + g_id
            return 0

        lax.fori_loop(0, CROWS * 8, idx_body, 0, unroll=True)
        for r in range(CROWS):
            pltpu.async_copy(xbs16.at[gi.at[P, r]],
                             rows.at[P, pl.ds(r * 128, 128), :], gsem.at[P])

    def compute(P):
        for r in range(CROWS):
            pltpu.make_async_copy(xbs16.at[gi.at[P, r]],
                                  rows.at[P, pl.ds(r * 128, 128), :],
                                  gsem.at[P]).wait()

        def step_body(i, _):
            sel = 2 * i + khigh
            selp = 2 * i + klow
            dstv = plsc.load_gather(d_idx.at[P], [sel])
            dstvp = plsc.load_gather(d_idx.at[P], [selp])
            collide = dstv == dstvp
            v = plsc.load_gather(rows.at[P], [sel, k7])
            vp = plsc.load_gather(rows.at[P], [selp, k7])
            v2 = jnp.where(collide, jnp.maximum(v, vp), v)
            aidx = dstv * 8 + k7
            cur = plsc.load_gather(acc, [aidx])
            plsc.store_scatter(acc, [aidx], jnp.maximum(cur, v2))
            return 0

        lax.fori_loop(0, _NSTEP, step_body, 0)

    issue(0, 0)
    launch(0)
    issue(1, 1)

    def pair_body(k, _):
        c2 = 2 * k + 2
        c3 = 2 * k + 3
        launch(1)
        compute(0)

        @pl.when(c2 < _NCH_D)
        def _():
            issue(c2, 0)
            launch(0)

        compute(1)

        @pl.when(c3 < _NCH_D)
        def _():
            issue(c3, 1)

        return 0

    lax.fori_loop(0, _NCH_D // 2, pair_body, 0)
    pltpu.sync_copy(acc, out.at[sh_id, g_id])


def _assemble_body(x_ref, xas_ref, cv_ref, m0_ref, m1_ref, out_ref):
    m = jnp.maximum(m0_ref[...], m1_ref[...])
    h = xas_ref[...] + cv_ref[...] + m
    out_ref[...] = jnp.maximum(h, 0.0) + x_ref[...]


def _assemble(x, xas, cv, m0, m1):
    return pl.pallas_call(
        _assemble_body,
        out_shape=jax.ShapeDtypeStruct((N, D), _f32),
    )(x, xas, cv, m0, m1)


def _block(x, srcs, dsts, W, b, g, be):
    xa, xb = _precompute(x, W)
    xa8 = xa.reshape(N * 8, 16)
    xb8 = xb.reshape(N * 8, 16)
    ss = _stats_kernel(xa8, xb8, srcs, dsts)
    ss = ss.reshape(4, 8, 2, 16).transpose(0, 2, 1, 3).reshape(4, 2, D)
    xbs, xas, cv = _scale2(ss, b, g, be, xb, xa)
    xbs16 = xbs.reshape(N * 16, 8)
    mm = _segmax_kernel(xbs16, srcs, dsts)
    mm = mm.reshape(2, 16, N, 8).transpose(0, 2, 1, 3).reshape(2, N, D)
    return _assemble(x, xas, cv, mm[0], mm[1])


def kernel(x, edge_index, W1, b1, g1, be1, W2, b2, g2, be2):
    srcs = edge_index[0]
    dsts = edge_index[1]
    x1 = _block(x, srcs, dsts, W1, b1, g1, be1)
    x2 = _block(x1, srcs, dsts, W2, b2, g2, be2)
    return x2
